# Initial kernel scaffold; baseline (speedup 1.0000x reference)
import jax
import jax.numpy as jnp
from jax.experimental import pallas as pl


def kernel(x, e, edge_index, att_params, node_params):
    raise NotImplementedError("write your pallas kernel here")


# speedup vs baseline: 1.4622x; 1.4622x over previous
import functools

import jax
import jax.numpy as jnp
from jax import lax
from jax.experimental import pallas as pl
from jax.experimental.pallas import tpu as pltpu
from jax.experimental.pallas import tpu_sc as plsc

_EPS = 1e-5

_NC = 2
_NS = 16
_NW = _NC * _NS

_CH = 40
_G_NCH = 125
_S_NCH = 125


def _ln_relu(z, g, be):
    m = jnp.mean(z, axis=-1, keepdims=True)
    v = jnp.mean((z - m) ** 2, axis=-1, keepdims=True)
    return jnp.maximum((z - m) / jnp.sqrt(v + _EPS) * g + be, 0.0)


def _xa_body(x_ref, w_ref, o_ref):
    o_ref[...] = jnp.dot(x_ref[...], w_ref[...])


def _node_space_matmul(x, w1b):
    n, d = x.shape
    blk = 1000
    return pl.pallas_call(
        _xa_body,
        grid=(n // blk,),
        in_specs=[
            pl.BlockSpec((blk, d), lambda i: (i, 0)),
            pl.BlockSpec((d, d), lambda i: (0, 0)),
        ],
        out_specs=pl.BlockSpec((blk, d), lambda i: (i, 0)),
        out_shape=jax.ShapeDtypeStruct((n, d), jnp.float32),
    )(x, w1b)


def _edge_body(e_ref, xg_ref, w1a, b1, g1, be1, w2, b2, g2, be2,
               w3, b3, g3, be3, o_ref):
    z = jnp.dot(e_ref[...], w1a[...]) + xg_ref[...] + b1[...]
    h = _ln_relu(z, g1[...], be1[...])
    h = _ln_relu(jnp.dot(h, w2[...]) + b2[...], g2[...], be2[...])
    h = _ln_relu(jnp.dot(h, w3[...]) + b3[...], g3[...], be3[...])
    o_ref[...] = h


def _edge_mlp(e, xg, w1a, p1, p2, p3):
    ecount, d = e.shape
    blk = 1280
    row = lambda i: (i, 0)
    zero = lambda i: (0, 0)
    wspec = pl.BlockSpec((d, d), zero)
    vspec = pl.BlockSpec((1, d), zero)
    b1, g1, be1 = p1
    w2, b2, g2, be2 = p2
    w3, b3, g3, be3 = p3
    vecs = [v.reshape(1, d) for v in (b1, g1, be1)]
    vecs2 = [v.reshape(1, d) for v in (b2, g2, be2)]
    vecs3 = [v.reshape(1, d) for v in (b3, g3, be3)]
    return pl.pallas_call(
        _edge_body,
        grid=(ecount // blk,),
        in_specs=[
            pl.BlockSpec((blk, d), row),
            pl.BlockSpec((blk, d), row),
            wspec, vspec, vspec, vspec,
            wspec, vspec, vspec, vspec,
            wspec, vspec, vspec, vspec,
        ],
        out_specs=pl.BlockSpec((blk, d), row),
        out_shape=jax.ShapeDtypeStruct((ecount, d), jnp.float32),
    )(e, xg, w1a, *vecs, w2, *vecs2, w3, *vecs3)


def _node_body(mi_ref, x_ref, v1a, v1b, c1, g1, be1, v2, b2, g2, be2,
               v3, b3, g3, be3, o_ref):
    z = (jnp.dot(mi_ref[...], v1a[...]) + jnp.dot(x_ref[...], v1b[...])
         + c1[...])
    h = _ln_relu(z, g1[...], be1[...])
    h = _ln_relu(jnp.dot(h, v2[...]) + b2[...], g2[...], be2[...])
    h = _ln_relu(jnp.dot(h, v3[...]) + b3[...], g3[...], be3[...])
    o_ref[...] = h


def _node_mlp(mi_pad, x, node_params):
    n, d = x.shape
    blk = 1000
    row = lambda i: (i, 0)
    zero = lambda i: (0, 0)
    wspec = pl.BlockSpec((d, d), zero)
    vspec = pl.BlockSpec((1, d), zero)
    (v1, c1, g1, be1), (v2, b2, g2, be2), (v3, b3, g3, be3) = node_params
    v1a = v1[:d]
    v1b = v1[d:]
    vecs = [t.reshape(1, d) for t in (c1, g1, be1)]
    vecs2 = [t.reshape(1, d) for t in (b2, g2, be2)]
    vecs3 = [t.reshape(1, d) for t in (b3, g3, be3)]
    return pl.pallas_call(
        _node_body,
        grid=(n // blk,),
        in_specs=[
            pl.BlockSpec((blk, d), row),
            pl.BlockSpec((blk, d), row),
            wspec, wspec, vspec, vspec, vspec,
            wspec, vspec, vspec, vspec,
            wspec, vspec, vspec, vspec,
        ],
        out_specs=pl.BlockSpec((blk, d), row),
        out_shape=jax.ShapeDtypeStruct((n, d), jnp.float32),
    )(mi_pad, x, v1a, v1b, *vecs, v2, *vecs2, v3, *vecs3)


def _sc_gather(table, idx3):
    nw, nch, ch = idx3.shape
    n, d = table.shape
    ecount = nw * nch * ch
    mesh = plsc.VectorSubcoreMesh(core_axis_name="c", subcore_axis_name="s")

    @functools.partial(
        pl.kernel,
        out_type=jax.ShapeDtypeStruct((ecount, d), jnp.float32),
        mesh=mesh,
        scratch_types=[
            pltpu.VMEM((nch, ch), jnp.int32),
            pltpu.VMEM((ch, d), jnp.float32),
            pltpu.SemaphoreType.DMA,
        ],
    )
    def gk(table_hbm, idx_hbm, out_hbm, idx_v, buf, sem):
        wid = lax.axis_index("c") * _NS + lax.axis_index("s")
        pltpu.sync_copy(idx_hbm.at[wid], idx_v)
        base = wid * nch * ch

        def body(j, carry):
            pltpu.async_copy(table_hbm.at[idx_v.at[j]], buf, sem).wait()
            pltpu.sync_copy(buf, out_hbm.at[pl.ds(base + j * ch, ch)])
            return carry

        lax.fori_loop(0, nch, body, 0)

    return gk(table, idx3)


_OWN = 320
_STRIP = 4000
_GCH = 48


def _sc_scatter_add(src, end_idx, zeros):
    ecount, d = src.shape
    nstrip = ecount // _STRIP
    nq = _STRIP // 16
    mesh = plsc.VectorSubcoreMesh(core_axis_name="c", subcore_axis_name="s")

    @functools.partial(
        pl.kernel,
        out_type=jax.ShapeDtypeStruct((_NW * _OWN, d), jnp.float32),
        mesh=mesh,
        compiler_params=pltpu.CompilerParams(needs_layout_passes=False),
        scratch_types=[
            pltpu.VMEM((_OWN, d), jnp.float32),
            pltpu.VMEM((_STRIP,), jnp.int32),
            pltpu.VMEM((_STRIP + 32,), jnp.int32),
            pltpu.VMEM((_STRIP + 32,), jnp.int32),
            pltpu.VMEM((_GCH, d), jnp.float32),
            pltpu.SemaphoreType.DMA,
        ],
    )
    def sk(src_hbm, end_hbm, z_hbm, out_hbm, accum, end_v, eid_l, dloc_l,
           buf, sem):
        w = lax.axis_index("c") * _NS + lax.axis_index("s")
        lo = w * _OWN
        pltpu.sync_copy(z_hbm, accum)
        iota = lax.iota(jnp.int32, 16)

        def init_body(q, c):
            eid_l[pl.ds(q * 16, 16)] = jnp.zeros((16,), jnp.int32)
            return c

        lax.fori_loop(0, (_STRIP + 32) // 16, init_body, 0)

        def strip_body(t, carry0):
            pltpu.sync_copy(end_hbm.at[pl.ds(t * _STRIP, _STRIP)], end_v)

            def scan_body(q, n):
                ev = end_v[pl.ds(q * 16, 16)]
                dl = ev - jnp.full((16,), lo, jnp.int32)
                mask = (dl >= jnp.zeros((16,), jnp.int32)) & (
                    dl < jnp.full((16,), _OWN, jnp.int32))
                mi32 = jnp.where(mask, jnp.full((16,), 1, jnp.int32),
                                 jnp.zeros((16,), jnp.int32))
                pos = jnp.full((16,), n - 1, jnp.int32) + plsc.cumsum(mi32)
                eidv = jnp.full((16,), t * _STRIP + q * 16, jnp.int32) + iota
                plsc.store_scatter(eid_l, [pos], eidv, mask=mask)
                plsc.store_scatter(dloc_l, [pos], dl, mask=mask)
                return lax.reduce_max(pos, (0,)) + 1

            n = lax.fori_loop(0, nq, scan_body, 0)
            nk = (n + _GCH - 1) // _GCH

            def chunk_body(k, carry1):
                pltpu.async_copy(
                    src_hbm.at[eid_l.at[pl.ds(k * _GCH, _GCH)]], buf, sem
                ).wait()
                lim = jnp.minimum(_GCH, n - k * _GCH)

                def edge_body(i, carry2):
                    dls = plsc.load_gather(
                        dloc_l, [jnp.full((16,), k * _GCH + i, jnp.int32)])
                    for kk in range(d // 16):
                        val = buf[i, pl.ds(kk * 16, 16)]
                        plsc.addupdate_scatter(
                            accum, [dls, kk * 16 + iota], val)
                    return carry2

                lax.fori_loop(0, lim, edge_body, 0)
                return carry1

            lax.fori_loop(0, nk, chunk_body, 0)
            return carry0

        lax.fori_loop(0, nstrip, strip_body, 0)
        pltpu.sync_copy(accum, out_hbm.at[pl.ds(lo, _OWN)])

    return sk(src, end_idx, zeros)


def kernel(x, e, edge_index, att_params, node_params):
    n, d = x.shape
    ecount = e.shape[0]
    start = edge_index[0]
    end = edge_index[1]
    (w1, b1, g1, be1), (w2, b2, g2, be2), (w3, b3, g3, be3) = att_params

    w1a = w1[:d]
    w1b = w1[d:]

    xa = _node_space_matmul(x, w1b)

    idx_g = start.reshape(_NW, _G_NCH, _CH)
    xg = _sc_gather(xa, idx_g)

    src = _edge_mlp(e, xg, w1a, (b1, g1, be1), (w2, b2, g2, be2),
                    (w3, b3, g3, be3))

    zeros = jnp.zeros((_OWN, d), jnp.float32)
    mi_pad = _sc_scatter_add(src, end, zeros)

    return _node_mlp(mi_pad, x, node_params)
